# baseline (device time: 160037 ns/iter reference)
import jax
import jax.numpy as jnp
from jax import lax
from jax.experimental import pallas as pl
from jax.experimental.pallas import tpu as pltpu

N_DEV = 4
M = 2048
N = 2048
M_CH = M // N_DEV
M_H = M_CH // 2
K_SUB = 4
M_S = M_H // K_SUB
N_HOP = N_DEV - 1


def kernel(x, w_mat):
    def body(x_hbm, w_hbm, out_hbm, x_ref, w_ref, y_ref, buf_cw, buf_ccw,
             send_cw, recv_cw, send_ccw, recv_ccw, store_sems, load_sems):
        my = lax.axis_index("i")
        left = (my - 1) % N_DEV
        right = (my + 1) % N_DEV

        w_load = pltpu.make_async_copy(w_hbm, w_ref, load_sems.at[0])
        w_load.start()
        x_loads = []
        for o in range(N_DEV):
            c = (my + o) % N_DEV
            r = pl.ds(c * M_CH, M_CH)
            cp = pltpu.make_async_copy(
                x_hbm.at[r, :], x_ref.at[r, :], load_sems.at[1 + o]
            )
            cp.start()
            x_loads.append(cp)


        def a_rows(c, j):
            return pl.ds(c * M_CH + j * M_S, M_S)

        def b_rows(c, j):
            return pl.ds(c * M_CH + M_H + j * M_S, M_S)

        def ch_rows(c):
            return pl.ds(c * M_CH, M_CH)

        def rs_cw(s, j):
            return pltpu.make_async_remote_copy(
                src_ref=y_ref.at[a_rows((my - s) % N_DEV, j), :],
                dst_ref=buf_cw.at[s * K_SUB + j],
                send_sem=send_cw.at[s * K_SUB + j],
                recv_sem=recv_cw.at[s * K_SUB + j],
                device_id=(right,),
                device_id_type=pl.DeviceIdType.MESH,
            )

        def rs_ccw(s, j):
            return pltpu.make_async_remote_copy(
                src_ref=y_ref.at[b_rows((my + s) % N_DEV, j), :],
                dst_ref=buf_ccw.at[s * K_SUB + j],
                send_sem=send_ccw.at[s * K_SUB + j],
                recv_sem=recv_ccw.at[s * K_SUB + j],
                device_id=(left,),
                device_id_type=pl.DeviceIdType.MESH,
            )

        def ag_cw(t, j):
            c = (my + 1 - t) % N_DEV
            return pltpu.make_async_remote_copy(
                src_ref=y_ref.at[a_rows(c, j), :],
                dst_ref=y_ref.at[a_rows(c, j), :],
                send_sem=send_cw.at[(N_HOP + t) * K_SUB + j],
                recv_sem=recv_cw.at[(N_HOP + t) * K_SUB + j],
                device_id=(right,),
                device_id_type=pl.DeviceIdType.MESH,
            )

        def ag_ccw(t, j):
            c = (my - 1 + t) % N_DEV
            return pltpu.make_async_remote_copy(
                src_ref=y_ref.at[b_rows(c, j), :],
                dst_ref=y_ref.at[b_rows(c, j), :],
                send_sem=send_ccw.at[(N_HOP + t) * K_SUB + j],
                recv_sem=recv_ccw.at[(N_HOP + t) * K_SUB + j],
                device_id=(left,),
                device_id_type=pl.DeviceIdType.MESH,
            )

        pending_sends = []
        pending_stores = []
        n_stores = [0]

        def launch(d):
            d.start()
            pending_sends.append(d)
            return d

        def store(rows):
            cp = pltpu.make_async_copy(
                y_ref.at[rows, :], out_hbm.at[rows, :],
                store_sems.at[n_stores[0]],
            )
            n_stores[0] += 1
            cp.start()
            pending_stores.append(cp)

        w_load.wait()
        x_loads[0].wait()
        cw = [None] * K_SUB
        ccw = [None] * K_SUB
        for j in range(K_SUB):
            r = a_rows(my, j)
            y_ref[r, :] = jnp.dot(
                x_ref[r, :], w_ref[:, :], preferred_element_type=jnp.float32
            )
            cw[j] = launch(rs_cw(0, j))
            r = b_rows(my, j)
            y_ref[r, :] = jnp.dot(
                x_ref[r, :], w_ref[:, :], preferred_element_type=jnp.float32
            )
            ccw[j] = launch(rs_ccw(0, j))

        for o in range(1, N_DEV):
            c = (my + o) % N_DEV
            x_loads[o].wait()
            y_ref[ch_rows(c), :] = jnp.dot(
                x_ref[ch_rows(c), :], w_ref[:, :],
                preferred_element_type=jnp.float32,
            )

        for s in range(N_HOP - 1):
            for j in range(K_SUB):
                cw[j].wait_recv()
                y_ref[a_rows((my - s - 1) % N_DEV, j), :] += buf_cw[s * K_SUB + j]
                cw[j] = launch(rs_cw(s + 1, j))
                ccw[j].wait_recv()
                y_ref[b_rows((my + s + 1) % N_DEV, j), :] += buf_ccw[s * K_SUB + j]
                ccw[j] = launch(rs_ccw(s + 1, j))

        s = N_HOP - 1
        ca = (my + 1) % N_DEV
        cb = (my - 1) % N_DEV
        for j in range(K_SUB):
            cw[j].wait_recv()
            ya = y_ref[a_rows(ca, j), :] + buf_cw[s * K_SUB + j]
            y_ref[a_rows(ca, j), :] = ya * jax.nn.sigmoid(ya)
            cw[j] = launch(ag_cw(0, j))
            store(a_rows(ca, j))
            ccw[j].wait_recv()
            yb = y_ref[b_rows(cb, j), :] + buf_ccw[s * K_SUB + j]
            y_ref[b_rows(cb, j), :] = yb * jax.nn.sigmoid(yb)
            ccw[j] = launch(ag_ccw(0, j))
            store(b_rows(cb, j))

        for t in range(N_HOP):
            for j in range(K_SUB):
                cw[j].wait_recv()
                if t < N_HOP - 1:
                    cw[j] = launch(ag_cw(t + 1, j))
                store(a_rows((my - t) % N_DEV, j))
                ccw[j].wait_recv()
                if t < N_HOP - 1:
                    ccw[j] = launch(ag_ccw(t + 1, j))
                store(b_rows((my + t) % N_DEV, j))

        for d in pending_sends:
            d.wait_send()
        for cp in pending_stores:
            cp.wait()

    n_slots = 2 * N_HOP * K_SUB
    n_store_slots = 2 * N_DEV * K_SUB
    return pl.pallas_call(
        body,
        out_shape=jax.ShapeDtypeStruct((M, N), jnp.float32),
        in_specs=[
            pl.BlockSpec(memory_space=pl.MemorySpace.ANY),
            pl.BlockSpec(memory_space=pl.MemorySpace.ANY),
        ],
        out_specs=pl.BlockSpec(memory_space=pl.MemorySpace.ANY),
        scratch_shapes=[
            pltpu.VMEM((M, x.shape[1]), jnp.float32),
            pltpu.VMEM((x.shape[1], N), jnp.float32),
            pltpu.VMEM((M, N), jnp.float32),
            pltpu.VMEM((N_HOP * K_SUB, M_S, N), jnp.float32),
            pltpu.VMEM((N_HOP * K_SUB, M_S, N), jnp.float32),
            pltpu.SemaphoreType.DMA((n_slots,)),
            pltpu.SemaphoreType.DMA((n_slots,)),
            pltpu.SemaphoreType.DMA((n_slots,)),
            pltpu.SemaphoreType.DMA((n_slots,)),
            pltpu.SemaphoreType.DMA((n_store_slots,)),
            pltpu.SemaphoreType.DMA((1 + N_DEV,)),
        ],
        compiler_params=pltpu.CompilerParams(
            vmem_limit_bytes=64 * 1024 * 1024,
        ),
    )(x, w_mat)


# device time: 156696 ns/iter; 1.0213x vs baseline; 1.0213x over previous
import jax
import jax.numpy as jnp
from jax import lax
from jax.experimental import pallas as pl
from jax.experimental.pallas import tpu as pltpu

N_DEV = 4
M = 2048
N = 2048
M_CH = M // N_DEV
M_H = M_CH // 2
K_SUB = 4
M_S = M_H // K_SUB
N_HOP = N_DEV - 1


def kernel(x, w_mat):
    def body(x_hbm, w_hbm, out_hbm, x_ref, w_ref, y_ref, buf_cw, buf_ccw,
             send_cw, recv_cw, send_ccw, recv_ccw, store_sems, load_sems):
        my = lax.axis_index("i")
        left = (my - 1) % N_DEV
        right = (my + 1) % N_DEV

        w_load = pltpu.make_async_copy(w_hbm, w_ref, load_sems.at[0])
        w_load.start()
        x_loads = []
        for o in range(N_DEV):
            c = (my + o) % N_DEV
            r = pl.ds(c * M_CH, M_CH)
            cp = pltpu.make_async_copy(
                x_hbm.at[r, :], x_ref.at[r, :], load_sems.at[1 + o]
            )
            cp.start()
            x_loads.append(cp)

        barrier_sem = pltpu.get_barrier_semaphore()
        for nbr in (left, right):
            pl.semaphore_signal(
                barrier_sem, inc=1,
                device_id=(nbr,), device_id_type=pl.DeviceIdType.MESH,
            )
        pl.semaphore_wait(barrier_sem, 2)

        def a_rows(c, j):
            return pl.ds(c * M_CH + j * M_S, M_S)

        def b_rows(c, j):
            return pl.ds(c * M_CH + M_H + j * M_S, M_S)

        def ch_rows(c):
            return pl.ds(c * M_CH, M_CH)

        def rs_cw(s, j):
            return pltpu.make_async_remote_copy(
                src_ref=y_ref.at[a_rows((my - s) % N_DEV, j), :],
                dst_ref=buf_cw.at[s * K_SUB + j],
                send_sem=send_cw.at[s * K_SUB + j],
                recv_sem=recv_cw.at[s * K_SUB + j],
                device_id=(right,),
                device_id_type=pl.DeviceIdType.MESH,
            )

        def rs_ccw(s, j):
            return pltpu.make_async_remote_copy(
                src_ref=y_ref.at[b_rows((my + s) % N_DEV, j), :],
                dst_ref=buf_ccw.at[s * K_SUB + j],
                send_sem=send_ccw.at[s * K_SUB + j],
                recv_sem=recv_ccw.at[s * K_SUB + j],
                device_id=(left,),
                device_id_type=pl.DeviceIdType.MESH,
            )

        def ag_cw(t, j):
            c = (my + 1 - t) % N_DEV
            return pltpu.make_async_remote_copy(
                src_ref=y_ref.at[a_rows(c, j), :],
                dst_ref=y_ref.at[a_rows(c, j), :],
                send_sem=send_cw.at[(N_HOP + t) * K_SUB + j],
                recv_sem=recv_cw.at[(N_HOP + t) * K_SUB + j],
                device_id=(right,),
                device_id_type=pl.DeviceIdType.MESH,
            )

        def ag_ccw(t, j):
            c = (my - 1 + t) % N_DEV
            return pltpu.make_async_remote_copy(
                src_ref=y_ref.at[b_rows(c, j), :],
                dst_ref=y_ref.at[b_rows(c, j), :],
                send_sem=send_ccw.at[(N_HOP + t) * K_SUB + j],
                recv_sem=recv_ccw.at[(N_HOP + t) * K_SUB + j],
                device_id=(left,),
                device_id_type=pl.DeviceIdType.MESH,
            )

        pending_sends = []
        pending_stores = []
        n_stores = [0]

        def launch(d):
            d.start()
            pending_sends.append(d)
            return d

        def store(rows):
            cp = pltpu.make_async_copy(
                y_ref.at[rows, :], out_hbm.at[rows, :],
                store_sems.at[n_stores[0]],
            )
            n_stores[0] += 1
            cp.start()
            pending_stores.append(cp)

        w_load.wait()
        x_loads[0].wait()
        cw = [None] * K_SUB
        ccw = [None] * K_SUB
        for j in range(K_SUB):
            r = a_rows(my, j)
            y_ref[r, :] = jnp.dot(
                x_ref[r, :], w_ref[:, :], preferred_element_type=jnp.float32
            )
            cw[j] = launch(rs_cw(0, j))
            r = b_rows(my, j)
            y_ref[r, :] = jnp.dot(
                x_ref[r, :], w_ref[:, :], preferred_element_type=jnp.float32
            )
            ccw[j] = launch(rs_ccw(0, j))

        for o in range(1, N_DEV):
            c = (my + o) % N_DEV
            x_loads[o].wait()
            y_ref[ch_rows(c), :] = jnp.dot(
                x_ref[ch_rows(c), :], w_ref[:, :],
                preferred_element_type=jnp.float32,
            )

        for s in range(N_HOP - 1):
            for j in range(K_SUB):
                cw[j].wait_recv()
                y_ref[a_rows((my - s - 1) % N_DEV, j), :] += buf_cw[s * K_SUB + j]
                cw[j] = launch(rs_cw(s + 1, j))
                ccw[j].wait_recv()
                y_ref[b_rows((my + s + 1) % N_DEV, j), :] += buf_ccw[s * K_SUB + j]
                ccw[j] = launch(rs_ccw(s + 1, j))

        s = N_HOP - 1
        ca = (my + 1) % N_DEV
        cb = (my - 1) % N_DEV
        for j in range(K_SUB):
            cw[j].wait_recv()
            ya = y_ref[a_rows(ca, j), :] + buf_cw[s * K_SUB + j]
            y_ref[a_rows(ca, j), :] = ya * jax.nn.sigmoid(ya)
            cw[j] = launch(ag_cw(0, j))
            store(a_rows(ca, j))
            ccw[j].wait_recv()
            yb = y_ref[b_rows(cb, j), :] + buf_ccw[s * K_SUB + j]
            y_ref[b_rows(cb, j), :] = yb * jax.nn.sigmoid(yb)
            ccw[j] = launch(ag_ccw(0, j))
            store(b_rows(cb, j))

        for t in range(N_HOP):
            for j in range(K_SUB):
                cw[j].wait_recv()
                if t < N_HOP - 1:
                    cw[j] = launch(ag_cw(t + 1, j))
                store(a_rows((my - t) % N_DEV, j))
                ccw[j].wait_recv()
                if t < N_HOP - 1:
                    ccw[j] = launch(ag_ccw(t + 1, j))
                store(b_rows((my + t) % N_DEV, j))

        for d in pending_sends:
            d.wait_send()
        for cp in pending_stores:
            cp.wait()

    n_slots = 2 * N_HOP * K_SUB
    n_store_slots = 2 * N_DEV * K_SUB
    return pl.pallas_call(
        body,
        out_shape=jax.ShapeDtypeStruct((M, N), jnp.float32),
        in_specs=[
            pl.BlockSpec(memory_space=pl.MemorySpace.ANY),
            pl.BlockSpec(memory_space=pl.MemorySpace.ANY),
        ],
        out_specs=pl.BlockSpec(memory_space=pl.MemorySpace.ANY),
        scratch_shapes=[
            pltpu.VMEM((M, x.shape[1]), jnp.float32),
            pltpu.VMEM((x.shape[1], N), jnp.float32),
            pltpu.VMEM((M, N), jnp.float32),
            pltpu.VMEM((N_HOP * K_SUB, M_S, N), jnp.float32),
            pltpu.VMEM((N_HOP * K_SUB, M_S, N), jnp.float32),
            pltpu.SemaphoreType.DMA((n_slots,)),
            pltpu.SemaphoreType.DMA((n_slots,)),
            pltpu.SemaphoreType.DMA((n_slots,)),
            pltpu.SemaphoreType.DMA((n_slots,)),
            pltpu.SemaphoreType.DMA((n_store_slots,)),
            pltpu.SemaphoreType.DMA((1 + N_DEV,)),
        ],
        compiler_params=pltpu.CompilerParams(
            collective_id=0,
            vmem_limit_bytes=64 * 1024 * 1024,
            skip_device_barrier=True,
        ),
    )(x, w_mat)


# device time: 156674 ns/iter; 1.0215x vs baseline; 1.0001x over previous
import jax
import jax.numpy as jnp
from jax import lax
from jax.experimental import pallas as pl
from jax.experimental.pallas import tpu as pltpu

N_DEV = 4
M = 2048
N = 2048
M_CH = M // N_DEV
M_H = M_CH // 2
K_SUB = 4
M_S = M_H // K_SUB
N_HOP = N_DEV - 1


def kernel(x, w_mat):
    def body(x_hbm, w_hbm, out_hbm, x_ref, w_ref, y_ref, buf_cw, buf_ccw,
             send_cw, recv_cw, send_ccw, recv_ccw, store_sems, load_sems):
        my = lax.axis_index("i")
        left = (my - 1) % N_DEV
        right = (my + 1) % N_DEV

        w_load = pltpu.make_async_copy(w_hbm, w_ref, load_sems.at[0])
        w_load.start()
        x_loads = []
        for o in range(N_DEV):
            c = (my + o) % N_DEV
            r = pl.ds(c * M_CH, M_CH)
            cp = pltpu.make_async_copy(
                x_hbm.at[r, :], x_ref.at[r, :], load_sems.at[1 + o]
            )
            cp.start()
            x_loads.append(cp)

        barrier_sem = pltpu.get_barrier_semaphore()
        for nbr in (left, right):
            pl.semaphore_signal(
                barrier_sem, inc=1,
                device_id=(nbr,), device_id_type=pl.DeviceIdType.MESH,
            )
        pl.semaphore_wait(barrier_sem, 2)

        def a_rows(c, j):
            return pl.ds(c * M_CH + j * M_S, M_S)

        def b_rows(c, j):
            return pl.ds(c * M_CH + M_H + j * M_S, M_S)

        def ch_rows(c):
            return pl.ds(c * M_CH, M_CH)

        def rs_cw(s, j):
            return pltpu.make_async_remote_copy(
                src_ref=y_ref.at[a_rows((my - s) % N_DEV, j), :],
                dst_ref=buf_cw.at[s * K_SUB + j],
                send_sem=send_cw.at[s * K_SUB + j],
                recv_sem=recv_cw.at[s * K_SUB + j],
                device_id=(right,),
                device_id_type=pl.DeviceIdType.MESH,
            )

        def rs_ccw(s, j):
            return pltpu.make_async_remote_copy(
                src_ref=y_ref.at[b_rows((my + s) % N_DEV, j), :],
                dst_ref=buf_ccw.at[s * K_SUB + j],
                send_sem=send_ccw.at[s * K_SUB + j],
                recv_sem=recv_ccw.at[s * K_SUB + j],
                device_id=(left,),
                device_id_type=pl.DeviceIdType.MESH,
            )

        def ag_cw(t, j):
            c = (my + 1 - t) % N_DEV
            return pltpu.make_async_remote_copy(
                src_ref=y_ref.at[a_rows(c, j), :],
                dst_ref=y_ref.at[a_rows(c, j), :],
                send_sem=send_cw.at[(N_HOP + t) * K_SUB + j],
                recv_sem=recv_cw.at[(N_HOP + t) * K_SUB + j],
                device_id=(right,),
                device_id_type=pl.DeviceIdType.MESH,
            )

        def ag_ccw(t, j):
            c = (my - 1 + t) % N_DEV
            return pltpu.make_async_remote_copy(
                src_ref=y_ref.at[b_rows(c, j), :],
                dst_ref=y_ref.at[b_rows(c, j), :],
                send_sem=send_ccw.at[(N_HOP + t) * K_SUB + j],
                recv_sem=recv_ccw.at[(N_HOP + t) * K_SUB + j],
                device_id=(left,),
                device_id_type=pl.DeviceIdType.MESH,
            )

        pending_sends = []
        pending_stores = []
        n_stores = [0]

        def launch(d):
            d.start()
            pending_sends.append(d)
            return d

        def store(rows):
            cp = pltpu.make_async_copy(
                y_ref.at[rows, :], out_hbm.at[rows, :],
                store_sems.at[n_stores[0]],
            )
            n_stores[0] += 1
            cp.start()
            pending_stores.append(cp)

        w_load.wait()
        x_loads[0].wait()
        cw = [None] * K_SUB
        ccw = [None] * K_SUB
        for j in range(K_SUB):
            r = a_rows(my, j)
            y_ref[r, :] = jnp.dot(
                x_ref[r, :], w_ref[:, :], preferred_element_type=jnp.float32
            )
            cw[j] = launch(rs_cw(0, j))
            r = b_rows(my, j)
            y_ref[r, :] = jnp.dot(
                x_ref[r, :], w_ref[:, :], preferred_element_type=jnp.float32
            )
            ccw[j] = launch(rs_ccw(0, j))

        for o in range(1, N_DEV):
            c = (my + o) % N_DEV
            x_loads[o].wait()
            y_ref[ch_rows(c), :] = jnp.dot(
                x_ref[ch_rows(c), :], w_ref[:, :],
                preferred_element_type=jnp.float32,
            )

        for s in range(N_HOP - 1):
            for j in range(K_SUB):
                cw[j].wait_recv()
                y_ref[a_rows((my - s - 1) % N_DEV, j), :] += buf_cw[s * K_SUB + j]
                cw[j] = launch(rs_cw(s + 1, j))
                ccw[j].wait_recv()
                y_ref[b_rows((my + s + 1) % N_DEV, j), :] += buf_ccw[s * K_SUB + j]
                ccw[j] = launch(rs_ccw(s + 1, j))

        s = N_HOP - 1
        ca = (my + 1) % N_DEV
        cb = (my - 1) % N_DEV
        for j in range(K_SUB):
            cw[j].wait_recv()
            ya = y_ref[a_rows(ca, j), :] + buf_cw[s * K_SUB + j]
            y_ref[a_rows(ca, j), :] = ya * jax.nn.sigmoid(ya)
            cw[j] = launch(ag_cw(0, j))
            store(a_rows(ca, j))
            ccw[j].wait_recv()
            yb = y_ref[b_rows(cb, j), :] + buf_ccw[s * K_SUB + j]
            y_ref[b_rows(cb, j), :] = yb * jax.nn.sigmoid(yb)
            ccw[j] = launch(ag_ccw(0, j))
            store(b_rows(cb, j))

        for t in range(N_HOP):
            for j in range(K_SUB):
                cw[j].wait_recv()
                if t < N_HOP - 1:
                    cw[j] = launch(ag_cw(t + 1, j))
                store(a_rows((my - t) % N_DEV, j))
                ccw[j].wait_recv()
                if t < N_HOP - 1:
                    ccw[j] = launch(ag_ccw(t + 1, j))
                store(b_rows((my + t) % N_DEV, j))

        for d in pending_sends:
            d.wait_send()
        for cp in pending_stores:
            cp.wait()

    n_slots = 2 * N_HOP * K_SUB
    n_store_slots = 2 * N_DEV * K_SUB
    return pl.pallas_call(
        body,
        out_shape=jax.ShapeDtypeStruct((M, N), jnp.float32),
        in_specs=[
            pl.BlockSpec(memory_space=pl.MemorySpace.ANY),
            pl.BlockSpec(memory_space=pl.MemorySpace.ANY),
        ],
        out_specs=pl.BlockSpec(memory_space=pl.MemorySpace.ANY),
        scratch_shapes=[
            pltpu.VMEM((M, x.shape[1]), jnp.float32),
            pltpu.VMEM((x.shape[1], N), jnp.float32),
            pltpu.VMEM((M, N), jnp.float32),
            pltpu.VMEM((N_HOP * K_SUB, M_S, N), jnp.float32),
            pltpu.VMEM((N_HOP * K_SUB, M_S, N), jnp.float32),
            pltpu.SemaphoreType.DMA((n_slots,)),
            pltpu.SemaphoreType.DMA((n_slots,)),
            pltpu.SemaphoreType.DMA((n_slots,)),
            pltpu.SemaphoreType.DMA((n_slots,)),
            pltpu.SemaphoreType.DMA((n_store_slots,)),
            pltpu.SemaphoreType.DMA((1 + N_DEV,)),
        ],
        compiler_params=pltpu.CompilerParams(
            collective_id=0,
            vmem_limit_bytes=64 * 1024 * 1024,
        ),
    )(x, w_mat)
